# baseline (speedup 1.0000x reference)
import functools

import jax
import jax.numpy as jnp
import numpy as np
from jax import lax
from jax.experimental import pallas as pl
from jax.experimental.pallas import tpu as pltpu
from jax.experimental.pallas import tpu_sc as plsc

_NC = 2
_NS = 16
_NW = _NC * _NS
_L = 16

_G = 128


@functools.lru_cache(maxsize=None)
def _build_sc_gather(vocab: int, embed: int, batch: int, tp: int):
    assert embed == _L and vocab % tp == 0
    n_chunks = batch // tp
    bw = batch // _NW
    wpi = n_chunks // bw
    xc = bw * tp
    n_g = bw // _G
    n_v = bw // _L
    vpg = _G // _L

    mesh = plsc.VectorSubcoreMesh(core_axis_name="c", subcore_axis_name="s")

    @functools.partial(
        pl.kernel,
        out_type=jax.ShapeDtypeStruct((tp, n_chunks * embed), jnp.float32),
        mesh=mesh,
        scratch_types=[
            pltpu.VMEM((xc,), jnp.int32),
            pltpu.VMEM((n_g, _G), jnp.int32),
            pltpu.VMEM((n_g, _G), jnp.int32),
            pltpu.VMEM((2, _G, tp * embed), jnp.float32),
            pltpu.VMEM((bw * embed,), jnp.float32),
            pltpu.SemaphoreType.DMA,
            pltpu.SemaphoreType.DMA,
        ],
        compiler_params=pltpu.CompilerParams(needs_layout_passes=False),
    )
    def k(x_hbm, tbl_hbm, out_hbm, xraw_v, blk_v, rem_v, g8_v, rv_v, s0, s1):
        sems = [s0, s1]
        wid = lax.axis_index("s") * _NC + lax.axis_index("c")
        i = wid // wpi
        j0 = (wid % wpi) * bw
        lanes = lax.iota(jnp.int32, _L)
        pltpu.sync_copy(x_hbm.at[pl.ds(j0 * tp, xc)], xraw_v)
        for v in range(n_v):
            vals = plsc.load_gather(xraw_v, [(lanes + v * _L) * tp + i])
            g, s = v // vpg, (v % vpg) * _L
            blk_v[g, pl.ds(s, _L)] = vals >> 3
            rem_v[g, pl.ds(s, _L)] = vals & 7
        def fire(g):
            return pltpu.async_copy(
                tbl_hbm.at[blk_v.at[np.int32(g)]],
                g8_v.at[np.int32(g % 2)],
                sems[g % 2],
            )
        cp = fire(0)
        for g in range(n_g):
            cp.wait()
            if g + 1 < n_g:
                cp = fire(g + 1)
            buf = lanes * 0 + (g % 2)
            for v in range(vpg):
                loc = lanes + v * _L
                rem = rem_v[g, pl.ds(v * _L, _L)]
                base = (loc + g * _G) * embed
                col = rem * embed
                for e in range(embed):
                    vals = plsc.load_gather(g8_v, [buf, loc, col + e])
                    plsc.store_scatter(rv_v, [base + e], vals)
        pltpu.sync_copy(rv_v, out_hbm.at[i, pl.ds(j0 * embed, bw * embed)])

    return k


def kernel(x, table):
    vocab, embed = table.shape
    (batch,) = x.shape
    tp = 8
    xi = x.astype(jnp.int32)
    tbl = table.astype(jnp.float32).reshape(vocab // tp, tp * embed)
    return _build_sc_gather(vocab, embed, batch, tp)(xi, tbl)


# speedup vs baseline: 1.5142x; 1.5142x over previous
import functools

import jax
import jax.numpy as jnp
import numpy as np
from jax import lax
from jax.experimental import pallas as pl
from jax.experimental.pallas import tpu as pltpu
from jax.experimental.pallas import tpu_sc as plsc

_NC = 2
_NS = 16
_NW = _NC * _NS
_L = 16

_PIECE = 32


@functools.lru_cache(maxsize=None)
def _build_sc_gather(vocab: int, embed: int, batch: int, tp: int):
    assert embed == _L and tp == 8
    n_chunks = batch // tp
    bw = batch // _NW
    wpi = n_chunks // bw
    xc = bw * tp
    n_p = bw // _PIECE

    mesh = plsc.VectorSubcoreMesh(core_axis_name="c", subcore_axis_name="s")

    @functools.partial(
        pl.kernel,
        out_type=jax.ShapeDtypeStruct((tp, n_chunks * embed), jnp.float32),
        mesh=mesh,
        scratch_types=[
            pltpu.VMEM((xc + _L,), jnp.int32),
            pltpu.VMEM((2, _PIECE * tp, embed), jnp.float32),
            pltpu.VMEM((bw * embed,), jnp.float32),
            pltpu.SemaphoreType.DMA,
            pltpu.SemaphoreType.DMA,
        ],
        compiler_params=pltpu.CompilerParams(needs_layout_passes=False),
    )
    def k(x_hbm, tbl_hbm, out_hbm, xraw_v, blk_v, rv_v, s0, s1):
        wid = lax.axis_index("s") * _NC + lax.axis_index("c")
        i = wid // wpi
        j0 = (wid % wpi) * bw
        lanes = lax.iota(jnp.int32, _L)
        zero = lanes * 0
        m_lo = lanes == i
        m_hi = lanes == i + tp
        pltpu.sync_copy(x_hbm.at[pl.ds(j0 * tp, xc)], xraw_v.at[pl.ds(0, xc)])

        rems = [None] * bw

        def fire(p, buf):
            for u2 in range(_PIECE // 2):
                xw = xraw_v[pl.ds((p * _PIECE + 2 * u2) * tp, _L)]
                for s, m in ((2 * u2, m_lo), (2 * u2 + 1, m_hi)):
                    xv = jnp.sum(jnp.where(m, xw, zero), dtype=jnp.int32)
                    a = pl.multiple_of(xv & ~7, tp)
                    rems[p * _PIECE + s] = xv & 7
                    pltpu.async_copy(
                        tbl_hbm.at[pl.ds(a, tp), :],
                        blk_v.at[np.int32(buf), pl.ds(s * tp, tp), :],
                        [s0, s1][buf],
                    )

        def drain_extract(p, buf):
            for u in range(_PIECE):
                pltpu.make_async_copy(
                    tbl_hbm.at[pl.ds(0, tp), :],
                    blk_v.at[np.int32(buf), pl.ds(u * tp, tp), :],
                    [s0, s1][buf],
                ).wait()
            for u in range(_PIECE):
                row = u * tp + rems[p * _PIECE + u]
                vec = blk_v[np.int32(buf), row, :]
                rv_v[pl.ds((p * _PIECE + u) * embed, embed)] = vec

        fire(0, 0)
        for p in range(n_p):
            if p + 1 < n_p:
                fire(p + 1, (p + 1) % 2)
            drain_extract(p, p % 2)

        pltpu.sync_copy(rv_v, out_hbm.at[i, pl.ds(j0 * embed, bw * embed)])

    return k


def kernel(x, table):
    vocab, embed = table.shape
    (batch,) = x.shape
    tp = 8
    xi = x.astype(jnp.int32)
    return _build_sc_gather(vocab, embed, batch, tp)(xi, table.astype(jnp.float32))
